# baseline (speedup 1.0000x reference)
import functools

import jax
import jax.numpy as jnp
from jax import lax
from jax.experimental import pallas as pl
from jax.experimental.pallas import tpu as pltpu
from jax.experimental.pallas import tpu_sc as plsc

NUM_ROWS = 1000000
D = 64
B = 16384
W = 128

NC = 2
NS = 16
L = 16
NW = NC * NS
BPW = B // NW
NSLOT = 6


def _mf_body(users_hbm, items_hbm, ut_hbm, it_hbm, out_hbm,
             sidx_u, sidx_i, vidx, buf_u, buf_i, out_v,
             sems_u, sems_i):
    wid = lax.axis_index("s") * NC + lax.axis_index("c")
    base = wid * BPW

    pltpu.sync_copy(users_hbm.at[pl.ds(base, BPW)], vidx)

    def spill_u(g, carry):
        v = vidx[pl.ds(g * L, L)]
        for i in range(L):
            sidx_u[g * L + i] = v[i]
        return carry

    lax.fori_loop(0, BPW // L, spill_u, 0)
    pltpu.sync_copy(items_hbm.at[pl.ds(base, BPW)], vidx)

    def spill_i(g, carry):
        v = vidx[pl.ds(g * L, L)]
        for i in range(L):
            sidx_i[g * L + i] = v[i]
        return carry

    lax.fori_loop(0, BPW // L, spill_i, 0)

    lane = lax.iota(jnp.int32, L)

    def fire(n, t):
        ru = sidx_u[n]
        ri = sidx_i[n]
        cu = pl.multiple_of(ru - ru, W)
        ci = pl.multiple_of(ri - ri, W)
        pltpu.async_copy(ut_hbm.at[pl.ds(0, D), pl.ds(cu, W)],
                         buf_u.at[t], sems_u.at[t])
        pltpu.async_copy(it_hbm.at[pl.ds(0, D), pl.ds(ci, W)],
                         buf_i.at[t], sems_i.at[t])

    def drain(t):
        pltpu.make_async_copy(ut_hbm.at[pl.ds(0, D), pl.ds(0, W)],
                              buf_u.at[t], sems_u.at[t]).wait()
        pltpu.make_async_copy(it_hbm.at[pl.ds(0, D), pl.ds(0, W)],
                              buf_i.at[t], sems_i.at[t]).wait()

    dnums = lax.GatherDimensionNumbers(
        offset_dims=(), collapsed_slice_dims=(0,), start_index_map=(0,))

    def permute(v, idx):
        return lax.gather(v, idx[:, None], dimension_numbers=dnums,
                          slice_sizes=(1,),
                          mode=lax.GatherScatterMode.PROMISE_IN_BOUNDS)

    def sum_lanes(v):
        for sh in (8, 4, 2, 1):
            v = v + permute(v, lane ^ sh)
        return v

    def lookup_dot(n, t):
        lu = jnp.broadcast_to(sidx_u[n] & (W - 1), (L,))
        li = jnp.broadcast_to(sidx_i[n] & (W - 1), (L,))
        if True:
            return buf_u[t, 0, pl.ds(0, L)] * buf_i[t, 0, pl.ds(0, L)]
        acc = None
        for c in range(D // L):
            fc = lane + (c * L)
            u = plsc.load_gather(buf_u.at[t], [fc, lu])
            v = plsc.load_gather(buf_i.at[t], [fc, li])
            acc = u * v if acc is None else acc + u * v
        return sum_lanes(acc)

    for t in range(NSLOT - 1):
        fire(t, t)

    def body(n, vec):
        t = lax.rem(n, NSLOT)
        drain(t)
        nn = n + (NSLOT - 1)

        @pl.when(nn < BPW)
        def _():
            fire(nn, lax.rem(nn, NSLOT))

        g = n & 15
        vec = jnp.where(lane == g, lookup_dot(n, t), vec)

        @pl.when(g == 15)
        def _():
            off = pl.multiple_of(((n >> 4) & 0xFFFFFF) * L, L)
            out_v[pl.ds(off, L)] = vec

        return jnp.where(g == 15, jnp.zeros((L,), jnp.float32), vec)

    lax.fori_loop(0, BPW, body, jnp.zeros((L,), jnp.float32))

    for t in range(BPW // L):
        x = out_v[pl.ds(t * L, L)]
        out_v[pl.ds(t * L, L)] = 1.0 / (1.0 + jnp.exp(-x))
    pltpu.sync_copy(out_v, out_hbm.at[pl.ds(base, BPW)])


@jax.jit
def kernel(users, items, user_table, item_table):
    mesh = plsc.VectorSubcoreMesh(core_axis_name="c", subcore_axis_name="s")
    run = pl.kernel(
        _mf_body,
        out_type=jax.ShapeDtypeStruct((B,), jnp.float32),
        mesh=mesh,
        compiler_params=pltpu.CompilerParams(needs_layout_passes=False),
        scratch_types=[
            pltpu.SMEM((BPW,), jnp.int32),
            pltpu.SMEM((BPW,), jnp.int32),
            pltpu.VMEM((BPW,), jnp.int32),
            pltpu.VMEM((NSLOT, D, W), jnp.float32),
            pltpu.VMEM((NSLOT, D, W), jnp.float32),
            pltpu.VMEM((BPW,), jnp.float32),
            pltpu.SemaphoreType.DMA((NSLOT,)),
            pltpu.SemaphoreType.DMA((NSLOT,)),
        ],
    )
    return run(users.astype(jnp.int32), items.astype(jnp.int32),
               user_table.T, item_table.T)


# speedup vs baseline: 6.2761x; 6.2761x over previous
import functools

import jax
import jax.numpy as jnp
from jax import lax
from jax.experimental import pallas as pl
from jax.experimental.pallas import tpu as pltpu
from jax.experimental.pallas import tpu_sc as plsc

NUM_ROWS = 1000000
D = 64
B = 16384
W = 128

NC = 2
NS = 16
L = 16
NW = NC * NS
BPW = B // NW
NSLOT = 6


def _mf_body(users_hbm, items_hbm, ut_hbm, it_hbm, out_hbm,
             sidx_u, sidx_i, vidx, buf_u, buf_i, out_v,
             sems_u, sems_i):
    wid = lax.axis_index("s") * NC + lax.axis_index("c")
    base = wid * BPW

    pltpu.sync_copy(users_hbm.at[pl.ds(base, BPW)], vidx)

    def spill_u(g, carry):
        v = vidx[pl.ds(g * L, L)]
        for i in range(L):
            sidx_u[g * L + i] = v[i]
        return carry

    lax.fori_loop(0, BPW // L, spill_u, 0)
    pltpu.sync_copy(items_hbm.at[pl.ds(base, BPW)], vidx)

    def spill_i(g, carry):
        v = vidx[pl.ds(g * L, L)]
        for i in range(L):
            sidx_i[g * L + i] = v[i]
        return carry

    lax.fori_loop(0, BPW // L, spill_i, 0)

    lane = lax.iota(jnp.int32, L)

    def fire(n, t):
        ru = sidx_u[n]
        ri = sidx_i[n]
        cu = pl.multiple_of((ru >> 7) << 7, W)
        ci = pl.multiple_of((ri >> 7) << 7, W)
        h = D // 2
        for p in range(2):
            pltpu.async_copy(ut_hbm.at[pl.ds(p * h, h), pl.ds(cu, W)],
                             buf_u.at[t, pl.ds(p * h, h)], sems_u.at[t])
            pltpu.async_copy(it_hbm.at[pl.ds(p * h, h), pl.ds(ci, W)],
                             buf_i.at[t, pl.ds(p * h, h)], sems_i.at[t])

    def drain(t):
        pltpu.make_async_copy(ut_hbm.at[pl.ds(0, D), pl.ds(0, W)],
                              buf_u.at[t], sems_u.at[t]).wait()
        pltpu.make_async_copy(it_hbm.at[pl.ds(0, D), pl.ds(0, W)],
                              buf_i.at[t], sems_i.at[t]).wait()

    dnums = lax.GatherDimensionNumbers(
        offset_dims=(), collapsed_slice_dims=(0,), start_index_map=(0,))

    def permute(v, idx):
        return lax.gather(v, idx[:, None], dimension_numbers=dnums,
                          slice_sizes=(1,),
                          mode=lax.GatherScatterMode.PROMISE_IN_BOUNDS)

    def sum_lanes(v):
        for sh in (8, 4, 2, 1):
            v = v + permute(v, lane ^ sh)
        return v

    def lookup_dot(n, t):
        lu = jnp.broadcast_to(sidx_u[n] & (W - 1), (L,))
        li = jnp.broadcast_to(sidx_i[n] & (W - 1), (L,))
        acc = None
        for c in range(D // L):
            fc = lane + (c * L)
            u = plsc.load_gather(buf_u.at[t], [fc, lu])
            v = plsc.load_gather(buf_i.at[t], [fc, li])
            acc = u * v if acc is None else acc + u * v
        return sum_lanes(acc)

    for t in range(NSLOT - 1):
        fire(t, t)

    def body(n, vec):
        t = lax.rem(n, NSLOT)
        drain(t)
        nn = n + (NSLOT - 1)

        @pl.when(nn < BPW)
        def _():
            fire(nn, lax.rem(nn, NSLOT))

        g = n & 15
        vec = jnp.where(lane == g, lookup_dot(n, t), vec)

        @pl.when(g == 15)
        def _():
            off = pl.multiple_of(((n >> 4) & 0xFFFFFF) * L, L)
            out_v[pl.ds(off, L)] = vec

        return jnp.where(g == 15, jnp.zeros((L,), jnp.float32), vec)

    lax.fori_loop(0, BPW, body, jnp.zeros((L,), jnp.float32))

    for t in range(BPW // L):
        x = out_v[pl.ds(t * L, L)]
        out_v[pl.ds(t * L, L)] = 1.0 / (1.0 + jnp.exp(-x))
    pltpu.sync_copy(out_v, out_hbm.at[pl.ds(base, BPW)])


@jax.jit
def kernel(users, items, user_table, item_table):
    mesh = plsc.VectorSubcoreMesh(core_axis_name="c", subcore_axis_name="s")
    run = pl.kernel(
        _mf_body,
        out_type=jax.ShapeDtypeStruct((B,), jnp.float32),
        mesh=mesh,
        compiler_params=pltpu.CompilerParams(needs_layout_passes=False),
        scratch_types=[
            pltpu.SMEM((BPW,), jnp.int32),
            pltpu.SMEM((BPW,), jnp.int32),
            pltpu.VMEM((BPW,), jnp.int32),
            pltpu.VMEM((NSLOT, D, W), jnp.float32),
            pltpu.VMEM((NSLOT, D, W), jnp.float32),
            pltpu.VMEM((BPW,), jnp.float32),
            pltpu.SemaphoreType.DMA((NSLOT,)),
            pltpu.SemaphoreType.DMA((NSLOT,)),
        ],
    )
    return run(users.astype(jnp.int32), items.astype(jnp.int32),
               user_table.T, item_table.T)
